# baseline (speedup 1.0000x reference)
import functools

import jax
import jax.numpy as jnp
from jax import lax
from jax.experimental import pallas as pl
from jax.experimental.pallas import tpu as pltpu
from jax.experimental.pallas import tpu_sc as plsc

D_MODEL_ = 64
SCALE_ = 8.0

_info = plsc.get_sparse_core_info()
_NC, _NS, _L = _info.num_cores, _info.num_subcores, _info.num_lanes
_NW = _NC * _NS


def _make_sc_embed(B, D, CHUNK):
    assert B % _NW == 0
    b_per_w = B // _NW
    assert b_per_w % CHUNK == 0
    n_chunks = b_per_w // CHUNK
    assert CHUNK % 8 == 0
    vregs_per_row = D // _L

    mesh = plsc.VectorSubcoreMesh(core_axis_name="c", subcore_axis_name="s")

    @functools.partial(
        pl.kernel,
        mesh=mesh,
        out_type=jax.ShapeDtypeStruct((B, D), jnp.float32),
        scratch_types=[
            pltpu.VMEM((CHUNK,), jnp.int32),
            pltpu.VMEM((CHUNK, D), jnp.float32),
            pltpu.SemaphoreType.DMA,
        ],
        compiler_params=pltpu.CompilerParams(use_tc_tiling_on_sc=False),
    )
    def k(idx_hbm, table_hbm, out_hbm, idx_v, rows_v, sem):
        wid = lax.axis_index("s") * _NC + lax.axis_index("c")
        base = wid * b_per_w

        def chunk_body(c, carry):
            row0 = base + c * CHUNK
            pltpu.sync_copy(idx_hbm.at[pl.ds(row0, CHUNK)], idx_v)
            pltpu.async_copy(table_hbm.at[idx_v], rows_v, sem).wait()

            def scale_body(j, carry2):
                for r in range(vregs_per_row):
                    sl = pl.ds(r * _L, _L)
                    rows_v[j, sl] = rows_v[j, sl] * SCALE_
                return carry2

            lax.fori_loop(0, CHUNK, scale_body, 0, unroll=4)
            pltpu.sync_copy(rows_v, out_hbm.at[pl.ds(row0, CHUNK)])
            return carry

        lax.fori_loop(0, n_chunks, chunk_body, 0)

    return k


@jax.jit
def kernel(x, table):
    B = x.shape[0] * x.shape[1]
    D = table.shape[1]
    flat_idx = x.reshape(-1).astype(jnp.int32)
    out = _make_sc_embed(B, D, 800)(flat_idx, table)
    return out.reshape(x.shape[0], x.shape[1], D)


# speedup vs baseline: 1.0225x; 1.0225x over previous
import functools

import jax
import jax.numpy as jnp
from jax import lax
from jax.experimental import pallas as pl
from jax.experimental.pallas import tpu as pltpu
from jax.experimental.pallas import tpu_sc as plsc

SCALE_ = 8.0

_info = plsc.get_sparse_core_info()
_NC, _NS, _L = _info.num_cores, _info.num_subcores, _info.num_lanes
_NW = _NC * _NS


def _make_sc_embed(B, D, CHUNK):
    assert B % _NW == 0
    b_per_w = B // _NW
    assert b_per_w % CHUNK == 0
    n_chunks = b_per_w // CHUNK
    assert CHUNK % 8 == 0
    vregs_per_row = D // _L

    mesh = plsc.VectorSubcoreMesh(core_axis_name="c", subcore_axis_name="s")

    @functools.partial(
        pl.kernel,
        mesh=mesh,
        out_type=jax.ShapeDtypeStruct((B, D), jnp.float32),
        scratch_types=[
            pltpu.VMEM((b_per_w,), jnp.int32),
            pltpu.VMEM((CHUNK, D), jnp.float32),
            pltpu.VMEM((CHUNK, D), jnp.float32),
            pltpu.SemaphoreType.DMA,
            pltpu.SemaphoreType.DMA,
            pltpu.SemaphoreType.DMA,
            pltpu.SemaphoreType.DMA,
        ],
        compiler_params=pltpu.CompilerParams(use_tc_tiling_on_sc=False),
    )
    def k(idx_hbm, table_hbm, out_hbm, idx_all, rows0, rows1, g0, g1, s0, s1):
        wid = lax.axis_index("s") * _NC + lax.axis_index("c")
        base = wid * b_per_w
        pltpu.sync_copy(idx_hbm.at[pl.ds(base, b_per_w)], idx_all)

        bufs = (rows0, rows1)
        gsem = (g0, g1)
        ssem = (s0, s1)

        def gather_start(cc):
            b = cc % 2
            return pltpu.async_copy(
                table_hbm.at[idx_all.at[pl.ds(cc * CHUNK, CHUNK)]],
                bufs[b],
                gsem[b],
            )

        def scatter_start(cc):
            b = cc % 2
            return pltpu.async_copy(
                bufs[b],
                out_hbm.at[pl.ds(base + cc * CHUNK, CHUNK)],
                ssem[b],
            )

        gathers = {0: gather_start(0)}
        scatters = {}
        for cc in range(n_chunks):
            b = cc % 2
            gathers[cc].wait()
            if cc >= 1:
                scatters[cc - 1].wait()
            if cc + 1 < n_chunks:
                gathers[cc + 1] = gather_start(cc + 1)

            buf = bufs[b]

            @plsc.parallel_loop(0, CHUNK, unroll=8)
            def _scale(j, _buf=buf):
                for r in range(vregs_per_row):
                    sl = pl.ds(r * _L, _L)
                    _buf[j, sl] = _buf[j, sl] * SCALE_

            scatters[cc] = scatter_start(cc)
        scatters[n_chunks - 1].wait()

    return k


@jax.jit
def kernel(x, table):
    B = x.shape[0] * x.shape[1]
    D = table.shape[1]
    flat_idx = x.reshape(-1).astype(jnp.int32)
    out = _make_sc_embed(B, D, 800)(flat_idx, table)
    return out.reshape(x.shape[0], x.shape[1], D)
